# baseline (speedup 1.0000x reference)
import functools
import jax
import jax.numpy as jnp
from jax import lax
from jax.experimental import pallas as pl
from jax.experimental.pallas import tpu as pltpu
from jax.experimental.pallas import tpu_sc as plsc

_NPOINT = 2048
_NS = 32
_EPS = 1e-5
_QB = 128


_N = 50000
_R = 391


def _fps_body(xp_ref, yp_ref, zp_ref, nx_ref, ny_ref, nz_ref, mind_ref):
    row_i = lax.broadcasted_iota(jnp.int32, (_R, 128), 0)
    lane_i = lax.broadcasted_iota(jnp.int32, (_R, 128), 1)
    flat_i = row_i * 128 + lane_i
    mind_ref[...] = jnp.where(flat_i < _N, jnp.float32(1e10), jnp.float32(-3e38))
    lane1 = lax.broadcasted_iota(jnp.int32, (1, 128), 1)

    def coord(ref, r, c):
        row = ref[pl.ds(r, 1), :]
        return jnp.sum(jnp.where(lane1 == c, row, jnp.float32(0.0)))

    lx0 = coord(xp_ref, 0, 0)
    ly0 = coord(yp_ref, 0, 0)
    lz0 = coord(zp_ref, 0, 0)
    nx_ref[0] = lx0
    ny_ref[0] = ly0
    nz_ref[0] = lz0

    def body(i, carry):
        lx, ly, lz = carry
        dx = xp_ref[...] - lx
        dy = yp_ref[...] - ly
        dz = zp_ref[...] - lz
        d = (jnp.abs(dx * dx) + jnp.abs(dy * dy)) + jnp.abs(dz * dz)
        mind = jnp.minimum(mind_ref[...], d)
        mind_ref[...] = mind
        m = jnp.max(mind)
        idx = jnp.min(jnp.where(mind == m, flat_i, jnp.int32(2 ** 30)))
        r = idx // 128
        c = idx - r * 128
        nlx = coord(xp_ref, r, c)
        nly = coord(yp_ref, r, c)
        nlz = coord(zp_ref, r, c)
        nx_ref[i] = nlx
        ny_ref[i] = nly
        nz_ref[i] = nlz
        return (nlx, nly, nlz)

    lax.fori_loop(1, _NPOINT, body, (lx0, ly0, lz0))


def _fps_pallas(planes):
    sspec = pl.BlockSpec(memory_space=pltpu.SMEM)
    nx, ny, nz = pl.pallas_call(
        _fps_body,
        in_specs=[pl.BlockSpec(memory_space=pltpu.VMEM)] * 3,
        out_specs=[sspec, sspec, sspec],
        out_shape=[jax.ShapeDtypeStruct((_NPOINT,), jnp.float32)] * 3,
        scratch_shapes=[pltpu.VMEM((_R, 128), jnp.float32)],
    )(*planes)
    return nx, ny, nz


def _fps_jax(xyz, npoint):
    N = xyz.shape[0]
    idxs = jnp.zeros((npoint,), dtype=jnp.int32)
    min_d = jnp.full((N,), 1e10, dtype=jnp.float32)

    def body(i, carry):
        min_d, idxs = carry
        last = xyz[idxs[i - 1]]
        d = jnp.sum((xyz - last) ** 2, axis=-1)
        min_d = jnp.minimum(min_d, d)
        idxs = idxs.at[i].set(jnp.argmax(min_d).astype(jnp.int32))
        return (min_d, idxs)

    _, idxs = lax.fori_loop(1, npoint, body, (min_d, idxs))
    return idxs


_P = _R * 128
_PCH = _R
_S = 48
_QK = 16


def _knn1_body(qx_ref, qy_ref, qz_ref, px_ref, py_ref, pz_ref, nc_ref):
    dx = qx_ref[...] - px_ref[...]
    dy = qy_ref[...] - py_ref[...]
    dz = qz_ref[...] - pz_ref[...]
    d = dx * dx + dy * dy + dz * dz
    cmin = jnp.min(d.reshape(_QK, _PCH, 128), axis=2)
    pad = jnp.full((_QK, 512 - _PCH), 3e38, jnp.float32)
    c = jnp.concatenate([cmin, pad], axis=1)
    li = lax.broadcasted_iota(jnp.int32, (_QK, 512), 1)
    cols = []
    for _ in range(_S):
        m = jnp.min(c, axis=1, keepdims=True)
        j = jnp.min(jnp.where(c == m, li, jnp.int32(10 ** 9)),
                    axis=1, keepdims=True)
        cols.append(j)
        c = jnp.where(li == j, jnp.float32(3e38), c)
    nc_ref[...] = jnp.concatenate(cols, axis=1)


def _knn_phase1(qx, qy, qz, px, py, pz):
    grid = _NPOINT // _QK
    qspec = pl.BlockSpec((_QK, 1), lambda i: (i, 0))
    pspec = pl.BlockSpec((1, _P), lambda i: (0, 0))
    return pl.pallas_call(
        _knn1_body,
        grid=(grid,),
        in_specs=[qspec, qspec, qspec, pspec, pspec, pspec],
        out_specs=pl.BlockSpec((_QK, _S), lambda i: (i, 0)),
        out_shape=jax.ShapeDtypeStruct((_NPOINT, _S), jnp.int32),
    )(qx[:, None], qy[:, None], qz[:, None], px, py, pz)


def _merge_low(ak, av, bk, bv):
    rbk = lax.rev(bk, (0,))
    rbv = lax.rev(bv, (0,))
    m = ak <= rbk
    lok = jnp.where(m, ak, rbk)
    lov = jnp.where(m, av, rbv)
    return plsc.sort_key_val(lok, lov)


def _merge_high(ak, av, bk, bv):
    rbk = lax.rev(bk, (0,))
    rbv = lax.rev(bv, (0,))
    m = ak <= rbk
    hik = jnp.where(m, rbk, ak)
    hiv = jnp.where(m, rbv, av)
    return plsc.sort_key_val(hik, hiv)


def _splat(ref, pos):
    g = (pos // 16) * 16
    reg = ref[pl.ds(g, 16)]
    dnums = lax.GatherDimensionNumbers(
        offset_dims=(), collapsed_slice_dims=(0,), start_index_map=(0,))
    return lax.gather(reg, jnp.full((16, 1), pos - g, jnp.int32), dnums,
                      (1,), mode=lax.GatherScatterMode.PROMISE_IN_BOUNDS)


def _knn_phase2(px2, py2, pz2, nx, ny, nz, ncflat):
    mesh = plsc.VectorSubcoreMesh(core_axis_name="c", subcore_axis_name="s")
    QW = _NPOINT // 32
    BIG = jnp.float32(3e38)

    @functools.partial(
        pl.kernel, mesh=mesh,
        out_type=jax.ShapeDtypeStruct((_NPOINT * 32,), jnp.int32),
        scratch_types=[
            pltpu.VMEM((128,), jnp.float32),
            pltpu.VMEM((128,), jnp.float32),
            pltpu.VMEM((128,), jnp.float32),
            pltpu.VMEM((_S,), jnp.int32),
            pltpu.VMEM((128,), jnp.int32),
            pltpu.VMEM((_S, 128), jnp.float32),
            pltpu.VMEM((_S, 128), jnp.float32),
            pltpu.VMEM((_S, 128), jnp.float32),
            pltpu.VMEM((32,), jnp.int32),
            pltpu.SemaphoreType.DMA,
        ],
    )
    def k(px_h, py_h, pz_h, nx_h, ny_h, nz_h, nc_h, out_h,
          qx_v, qy_v, qz_v, cid_v, cidp_v, xr_v, yr_v, zr_v, oi_v, sem):
        wid = lax.axis_index("s") * 2 + lax.axis_index("c")
        qbase = wid * QW
        pltpu.sync_copy(nx_h.at[pl.ds(qbase, QW)], qx_v.at[pl.ds(0, QW)])
        pltpu.sync_copy(ny_h.at[pl.ds(qbase, QW)], qy_v.at[pl.ds(0, QW)])
        pltpu.sync_copy(nz_h.at[pl.ds(qbase, QW)], qz_v.at[pl.ds(0, QW)])
        iota16 = lax.broadcasted_iota(jnp.int32, (16,), 0)

        def per_query(qi, _):
            q = qbase + qi
            pltpu.sync_copy(nc_h.at[pl.ds(q * _S, _S)], cid_v)
            pltpu.sync_copy(cid_v, cidp_v.at[pl.ds(0, _S)])
            pltpu.async_copy(px_h.at[cid_v], xr_v, sem).wait()
            pltpu.async_copy(py_h.at[cid_v], yr_v, sem).wait()
            pltpu.async_copy(pz_h.at[cid_v], zr_v, sem).wait()
            qx = _splat(qx_v, qi)
            qy = _splat(qy_v, qi)
            qz = _splat(qz_v, qi)

            def step(r, carry):
                t0k, t0v, t1k, t1v = carry
                j = r // 8
                s = (r - j * 8) * 16
                x = xr_v[j, pl.ds(s, 16)]
                y = yr_v[j, pl.ds(s, 16)]
                z = zr_v[j, pl.ds(s, 16)]
                ddx = x - qx
                ddy = y - qy
                ddz = z - qz
                d = ddx * ddx + ddy * ddy + ddz * ddz
                cid = _splat(cidp_v, j)
                idx = cid * 128 + s + iota16
                ck, cv = plsc.sort_key_val(d, idx)
                n0k, n0v = _merge_low(t0k, t0v, ck, cv)
                hk, hv = _merge_high(t0k, t0v, ck, cv)
                n1k, n1v = _merge_low(t1k, t1v, hk, hv)
                return n0k, n0v, n1k, n1v

            init = (jnp.full((16,), BIG), jnp.zeros((16,), jnp.int32),
                    jnp.full((16,), BIG), jnp.zeros((16,), jnp.int32))
            t0k, t0v, t1k, t1v = lax.fori_loop(0, _S * 8, step, init)
            oi_v[pl.ds(0, 16)] = t0v
            oi_v[pl.ds(16, 16)] = t1v
            pltpu.sync_copy(oi_v, out_h.at[pl.ds(q * 32, 32)])
            return 0

        lax.fori_loop(0, QW, per_query, 0)

    return k(px2, py2, pz2, nx, ny, nz, ncflat)


def _sc_gather(table, idx, chunk_rows):
    B = idx.shape[0]
    D = table.shape[1]
    bw = B // 32
    nch = bw // chunk_rows
    mesh = plsc.VectorSubcoreMesh(core_axis_name="c", subcore_axis_name="s")

    @functools.partial(
        pl.kernel, mesh=mesh,
        out_type=jax.ShapeDtypeStruct((B, D), jnp.float32),
        scratch_types=[
            pltpu.VMEM((chunk_rows,), jnp.int32),
            pltpu.VMEM((chunk_rows, D), jnp.float32),
            pltpu.SemaphoreType.DMA,
        ],
    )
    def k(tab_h, idx_h, out_h, idx_v, rows_v, sem):
        wid = lax.axis_index("s") * 2 + lax.axis_index("c")

        def body(i, _):
            base = wid * bw + i * chunk_rows
            pltpu.sync_copy(idx_h.at[pl.ds(base, chunk_rows)], idx_v)
            pltpu.async_copy(tab_h.at[idx_v], rows_v, sem).wait()
            pltpu.sync_copy(rows_v, out_h.at[pl.ds(base, chunk_rows)])
            return 0

        lax.fori_loop(0, nch, body, 0)

    return k(table, idx)


def _knn_jax(q, xyz, k, chunk=256):
    M = q.shape[0]
    qc = q.reshape(M // chunk, chunk, 3)

    def f(qb):
        d = jnp.sum((qb[:, None, :] - xyz[None, :, :]) ** 2, axis=-1)
        _, idx = lax.top_k(-d, k)
        return idx

    return lax.map(f, qc).reshape(M, k)


def _ln_relu(x, g, b):
    m = jnp.mean(x, axis=-1, keepdims=True)
    v = jnp.mean((x - m) ** 2, axis=-1, keepdims=True)
    return jnp.maximum((x - m) / jnp.sqrt(v + _EPS) * g + b, 0.0)


def _mlp_body(qe_ref, gx_ref, gf_ref,
              w0a_ref, w0b_ref, b0_ref, g0_ref, be0_ref,
              w1_ref, b1_ref, g1_ref, be1_ref,
              a1a_ref, a1b_ref, ab1_ref, a2_ref,
              pw_ref, pb_ref, ab2s_ref,
              out_ref):
    rel = gx_ref[...] - qe_ref[...]
    ff = gf_ref[...]

    w0a = w0a_ref[...]
    xw = rel[:, 0:1] * w0a[0:1, :]
    xw = xw + rel[:, 1:2] * w0a[1:2, :]
    xw = xw + rel[:, 2:3] * w0a[2:3, :]
    xw = xw + jnp.dot(ff, w0b_ref[...], preferred_element_type=jnp.float32)
    xw = xw + b0_ref[...]
    x = _ln_relu(xw, g0_ref[...], be0_ref[...])
    x = jnp.dot(x, w1_ref[...], preferred_element_type=jnp.float32) + b1_ref[...]
    x = _ln_relu(x, g1_ref[...], be1_ref[...])

    a1a = a1a_ref[...]
    a = rel[:, 0:1] * a1a[0:1, :]
    a = a + rel[:, 1:2] * a1a[1:2, :]
    a = a + rel[:, 2:3] * a1a[2:3, :]
    a = a + jnp.dot(x, a1b_ref[...], preferred_element_type=jnp.float32)
    a = jnp.maximum(a + ab1_ref[...], 0.0)
    s = jnp.dot(a, a2_ref[...], preferred_element_type=jnp.float32)
    s = s + ab2s_ref[0]

    s3 = s.reshape(_QB, _NS, 1)
    m3 = jnp.max(s3, axis=1, keepdims=True)
    e3 = jnp.exp(s3 - m3)
    w3 = e3 / jnp.sum(e3, axis=1, keepdims=True)
    x3 = x.reshape(_QB, _NS, 128)
    feats = jnp.sum(x3 * w3, axis=1)
    out_ref[...] = jnp.dot(feats, pw_ref[...],
                           preferred_element_type=jnp.float32) + pb_ref[...]


def _mlp_attn(qxyz, gxyz, gfeat, W0, b0, g0, be0, W1, b1, g1, be1,
              aW1, ab1, aW2, ab2, pW, pb):
    n = _NPOINT * _NS
    qe = jnp.repeat(qxyz, _NS, axis=0)
    qe4 = jnp.pad(qe, ((0, 0), (0, 1)))
    gx4 = jnp.pad(gxyz, ((0, 0), (0, 1)))
    w0a = jnp.pad(W0[:3], ((0, 1), (0, 0)))
    w0b = W0[3:]
    a1a = jnp.pad(aW1[:3], ((0, 1), (0, 0)))
    a1b = aW1[3:]
    b2 = jnp.broadcast_to

    grid = _NPOINT // _QB
    blk = _QB * _NS
    wspec = lambda shape: pl.BlockSpec(shape, lambda i: (0, 0))
    out = pl.pallas_call(
        _mlp_body,
        grid=(grid,),
        in_specs=[
            pl.BlockSpec((blk, 4), lambda i: (i, 0)),
            pl.BlockSpec((blk, 4), lambda i: (i, 0)),
            pl.BlockSpec((blk, 128), lambda i: (i, 0)),
            wspec((4, 128)), wspec((128, 128)),
            wspec((1, 128)), wspec((1, 128)), wspec((1, 128)),
            wspec((128, 128)), wspec((1, 128)), wspec((1, 128)), wspec((1, 128)),
            wspec((4, 128)), wspec((128, 128)), wspec((1, 128)),
            wspec((128, 1)),
            wspec((128, 128)), wspec((1, 128)),
            pl.BlockSpec(memory_space=pltpu.SMEM),
        ],
        out_specs=pl.BlockSpec((_QB, 128), lambda i: (i, 0)),
        out_shape=jax.ShapeDtypeStruct((_NPOINT, 128), jnp.float32),
    )(qe4, gx4, gfeat,
      w0a, w0b, b0[None, :], g0[None, :], be0[None, :],
      W1, b1[None, :], g1[None, :], be1[None, :],
      a1a, a1b, ab1[None, :],
      aW2,
      pW, pb[None, :], ab2)
    return out


def kernel(xyz, features, offset, W0, b0, g0, be0, W1, b1, g1, be1,
           aW1, ab1, aW2, ab2, pW, pb):
    pads = _P - _N
    planes = [jnp.pad(xyz[:, k], (0, pads), constant_values=1e6).reshape(_R, 128)
              for k in range(3)]
    nx, ny, nz = _fps_pallas(planes)
    new_xyz = jnp.stack([nx, ny, nz], axis=-1)
    nchunk = _knn_phase1(nx, ny, nz,
                         planes[0].reshape(1, _P), planes[1].reshape(1, _P),
                         planes[2].reshape(1, _P))
    cand = (nchunk[:, :, None] * 128
            + jnp.arange(128, dtype=jnp.int32)[None, None, :]).reshape(
                _NPOINT, _S * 128)
    pxyz = jnp.stack([planes[0].reshape(-1), planes[1].reshape(-1),
                      planes[2].reshape(-1)], axis=-1)
    cd = jnp.sum((pxyz[cand] - new_xyz[:, None, :]) ** 2, axis=-1)
    _, ci = lax.top_k(-cd, _NS)
    nidx = jnp.take_along_axis(cand, ci, axis=1).reshape(-1)
    gxyz = pxyz[nidx]
    gfeat = features[nidx]
    out = _mlp_attn(new_xyz, gxyz, gfeat, W0, b0, g0, be0,
                    W1, b1, g1, be1, aW1, ab1, aW2, ab2, pW, pb)
    new_offset = jnp.array([_NPOINT], dtype=jnp.int32)
    return new_xyz, out, new_offset


# speedup vs baseline: 4.9180x; 4.9180x over previous
import functools
import jax
import jax.numpy as jnp
from jax import lax
from jax.experimental import pallas as pl
from jax.experimental.pallas import tpu as pltpu
from jax.experimental.pallas import tpu_sc as plsc

_NPOINT = 2048
_NS = 32
_EPS = 1e-5
_QB = 128


_N = 50000
_R = 391


def _fps_body(xp_ref, yp_ref, zp_ref, nx_ref, ny_ref, nz_ref, mind_ref):
    row_i = lax.broadcasted_iota(jnp.int32, (_R, 128), 0)
    lane_i = lax.broadcasted_iota(jnp.int32, (_R, 128), 1)
    flat_i = row_i * 128 + lane_i
    mind_ref[...] = jnp.where(flat_i < _N, jnp.float32(1e10), jnp.float32(-3e38))
    lane1 = lax.broadcasted_iota(jnp.int32, (1, 128), 1)

    def coord(ref, r, c):
        row = ref[pl.ds(r, 1), :]
        return jnp.sum(jnp.where(lane1 == c, row, jnp.float32(0.0)))

    lx0 = coord(xp_ref, 0, 0)
    ly0 = coord(yp_ref, 0, 0)
    lz0 = coord(zp_ref, 0, 0)
    nx_ref[0] = lx0
    ny_ref[0] = ly0
    nz_ref[0] = lz0

    def body(i, carry):
        lx, ly, lz = carry
        dx = xp_ref[...] - lx
        dy = yp_ref[...] - ly
        dz = zp_ref[...] - lz
        d = (jnp.abs(dx * dx) + jnp.abs(dy * dy)) + jnp.abs(dz * dz)
        mind = jnp.minimum(mind_ref[...], d)
        mind_ref[...] = mind
        m = jnp.max(mind)
        idx = jnp.min(jnp.where(mind == m, flat_i, jnp.int32(2 ** 30)))
        r = idx // 128
        c = idx - r * 128
        nlx = coord(xp_ref, r, c)
        nly = coord(yp_ref, r, c)
        nlz = coord(zp_ref, r, c)
        nx_ref[i] = nlx
        ny_ref[i] = nly
        nz_ref[i] = nlz
        return (nlx, nly, nlz)

    lax.fori_loop(1, _NPOINT, body, (lx0, ly0, lz0))


def _fps_pallas(planes):
    sspec = pl.BlockSpec(memory_space=pltpu.SMEM)
    nx, ny, nz = pl.pallas_call(
        _fps_body,
        in_specs=[pl.BlockSpec(memory_space=pltpu.VMEM)] * 3,
        out_specs=[sspec, sspec, sspec],
        out_shape=[jax.ShapeDtypeStruct((_NPOINT,), jnp.float32)] * 3,
        scratch_shapes=[pltpu.VMEM((_R, 128), jnp.float32)],
    )(*planes)
    return nx, ny, nz


def _fps_jax(xyz, npoint):
    N = xyz.shape[0]
    idxs = jnp.zeros((npoint,), dtype=jnp.int32)
    min_d = jnp.full((N,), 1e10, dtype=jnp.float32)

    def body(i, carry):
        min_d, idxs = carry
        last = xyz[idxs[i - 1]]
        d = jnp.sum((xyz - last) ** 2, axis=-1)
        min_d = jnp.minimum(min_d, d)
        idxs = idxs.at[i].set(jnp.argmax(min_d).astype(jnp.int32))
        return (min_d, idxs)

    _, idxs = lax.fori_loop(1, npoint, body, (min_d, idxs))
    return idxs


_P = _R * 128
_PCH = _R
_S = 48
_QK = 16


def _knn1_body(qx_ref, qy_ref, qz_ref, px_ref, py_ref, pz_ref, d_ref):
    dx = qx_ref[...] - px_ref[...]
    dy = qy_ref[...] - py_ref[...]
    dz = qz_ref[...] - pz_ref[...]
    d_ref[...] = dx * dx + dy * dy + dz * dz


def _knn1_body_unused(qx_ref, qy_ref, qz_ref, px_ref, py_ref, pz_ref, nc_ref):
    dx = qx_ref[...] - px_ref[...]
    dy = qy_ref[...] - py_ref[...]
    dz = qz_ref[...] - pz_ref[...]
    d = dx * dx + dy * dy + dz * dz
    cmin = jnp.min(d.reshape(_QK, _PCH, 128), axis=2)
    pad = jnp.full((_QK, 512 - _PCH), 3e38, jnp.float32)
    c = jnp.concatenate([cmin, pad], axis=1)
    li = lax.broadcasted_iota(jnp.int32, (_QK, 512), 1)
    cols = []
    for _ in range(_S):
        m = jnp.min(c, axis=1, keepdims=True)
        j = jnp.min(jnp.where(c == m, li, jnp.int32(10 ** 9)),
                    axis=1, keepdims=True)
        cols.append(j)
        c = jnp.where(li == j, jnp.float32(3e38), c)
    nc_ref[...] = jnp.concatenate(cols, axis=1)


def _knn_phase1(qx, qy, qz, px, py, pz):
    grid = _NPOINT // _QK
    qspec = pl.BlockSpec((_QK, 1), lambda i: (i, 0))
    pspec = pl.BlockSpec((1, _P), lambda i: (0, 0))
    return pl.pallas_call(
        _knn1_body,
        grid=(grid,),
        in_specs=[qspec, qspec, qspec, pspec, pspec, pspec],
        out_specs=pl.BlockSpec((_QK, _P), lambda i: (i, 0)),
        out_shape=jax.ShapeDtypeStruct((_NPOINT, _P), jnp.float32),
    )(qx[:, None], qy[:, None], qz[:, None], px, py, pz)


def _merge_low(ak, av, bk, bv):
    rbk = lax.rev(bk, (0,))
    rbv = lax.rev(bv, (0,))
    m = ak <= rbk
    lok = jnp.where(m, ak, rbk)
    lov = jnp.where(m, av, rbv)
    return plsc.sort_key_val(lok, lov)


def _merge_high(ak, av, bk, bv):
    rbk = lax.rev(bk, (0,))
    rbv = lax.rev(bv, (0,))
    m = ak <= rbk
    hik = jnp.where(m, rbk, ak)
    hiv = jnp.where(m, rbv, av)
    return plsc.sort_key_val(hik, hiv)


def _splat(ref, pos):
    g = (pos // 16) * 16
    reg = ref[pl.ds(g, 16)]
    dnums = lax.GatherDimensionNumbers(
        offset_dims=(), collapsed_slice_dims=(0,), start_index_map=(0,))
    return lax.gather(reg, jnp.full((16, 1), pos - g, jnp.int32), dnums,
                      (1,), mode=lax.GatherScatterMode.PROMISE_IN_BOUNDS)


def _knn_phase2(px2, py2, pz2, nx, ny, nz, ncflat):
    mesh = plsc.VectorSubcoreMesh(core_axis_name="c", subcore_axis_name="s")
    QW = _NPOINT // 32
    BIG = jnp.float32(3e38)

    @functools.partial(
        pl.kernel, mesh=mesh,
        out_type=jax.ShapeDtypeStruct((_NPOINT * 32,), jnp.int32),
        scratch_types=[
            pltpu.VMEM((128,), jnp.float32),
            pltpu.VMEM((128,), jnp.float32),
            pltpu.VMEM((128,), jnp.float32),
            pltpu.VMEM((_S,), jnp.int32),
            pltpu.VMEM((128,), jnp.int32),
            pltpu.VMEM((_S, 128), jnp.float32),
            pltpu.VMEM((_S, 128), jnp.float32),
            pltpu.VMEM((_S, 128), jnp.float32),
            pltpu.VMEM((32,), jnp.int32),
            pltpu.SemaphoreType.DMA,
        ],
    )
    def k(px_h, py_h, pz_h, nx_h, ny_h, nz_h, nc_h, out_h,
          qx_v, qy_v, qz_v, cid_v, cidp_v, xr_v, yr_v, zr_v, oi_v, sem):
        wid = lax.axis_index("s") * 2 + lax.axis_index("c")
        qbase = wid * QW
        pltpu.sync_copy(nx_h.at[pl.ds(qbase, QW)], qx_v.at[pl.ds(0, QW)])
        pltpu.sync_copy(ny_h.at[pl.ds(qbase, QW)], qy_v.at[pl.ds(0, QW)])
        pltpu.sync_copy(nz_h.at[pl.ds(qbase, QW)], qz_v.at[pl.ds(0, QW)])
        iota16 = lax.broadcasted_iota(jnp.int32, (16,), 0)

        def per_query(qi, _):
            q = qbase + qi
            pltpu.sync_copy(nc_h.at[pl.ds(q * _S, _S)], cid_v)
            pltpu.sync_copy(cid_v, cidp_v.at[pl.ds(0, _S)])
            pltpu.async_copy(px_h.at[cid_v], xr_v, sem).wait()
            pltpu.async_copy(py_h.at[cid_v], yr_v, sem).wait()
            pltpu.async_copy(pz_h.at[cid_v], zr_v, sem).wait()
            qx = _splat(qx_v, qi)
            qy = _splat(qy_v, qi)
            qz = _splat(qz_v, qi)

            def step(r, carry):
                t0k, t0v, t1k, t1v = carry
                j = r // 8
                s = (r - j * 8) * 16
                x = xr_v[j, pl.ds(s, 16)]
                y = yr_v[j, pl.ds(s, 16)]
                z = zr_v[j, pl.ds(s, 16)]
                ddx = x - qx
                ddy = y - qy
                ddz = z - qz
                d = ddx * ddx + ddy * ddy + ddz * ddz
                cid = _splat(cidp_v, j)
                idx = cid * 128 + s + iota16
                ck, cv = plsc.sort_key_val(d, idx)
                n0k, n0v = _merge_low(t0k, t0v, ck, cv)
                hk, hv = _merge_high(t0k, t0v, ck, cv)
                n1k, n1v = _merge_low(t1k, t1v, hk, hv)
                return n0k, n0v, n1k, n1v

            init = (jnp.full((16,), BIG), jnp.zeros((16,), jnp.int32),
                    jnp.full((16,), BIG), jnp.zeros((16,), jnp.int32))
            t0k, t0v, t1k, t1v = lax.fori_loop(0, _S * 8, step, init)
            oi_v[pl.ds(0, 16)] = t0v
            oi_v[pl.ds(16, 16)] = t1v
            pltpu.sync_copy(oi_v, out_h.at[pl.ds(q * 32, 32)])
            return 0

        lax.fori_loop(0, QW, per_query, 0)

    return k(px2, py2, pz2, nx, ny, nz, ncflat)


def _sc_gather(table, idx, chunk_rows):
    B = idx.shape[0]
    D = table.shape[1]
    bw = B // 32
    nch = bw // chunk_rows
    mesh = plsc.VectorSubcoreMesh(core_axis_name="c", subcore_axis_name="s")

    @functools.partial(
        pl.kernel, mesh=mesh,
        out_type=jax.ShapeDtypeStruct((B, D), jnp.float32),
        scratch_types=[
            pltpu.VMEM((chunk_rows,), jnp.int32),
            pltpu.VMEM((chunk_rows, D), jnp.float32),
            pltpu.SemaphoreType.DMA,
        ],
    )
    def k(tab_h, idx_h, out_h, idx_v, rows_v, sem):
        wid = lax.axis_index("s") * 2 + lax.axis_index("c")

        def body(i, _):
            base = wid * bw + i * chunk_rows
            pltpu.sync_copy(idx_h.at[pl.ds(base, chunk_rows)], idx_v)
            pltpu.async_copy(tab_h.at[idx_v], rows_v, sem).wait()
            pltpu.sync_copy(rows_v, out_h.at[pl.ds(base, chunk_rows)])
            return 0

        lax.fori_loop(0, nch, body, 0)

    return k(table, idx)


def _knn_jax(q, xyz, k, chunk=256):
    M = q.shape[0]
    qc = q.reshape(M // chunk, chunk, 3)

    def f(qb):
        d = jnp.sum((qb[:, None, :] - xyz[None, :, :]) ** 2, axis=-1)
        _, idx = lax.top_k(-d, k)
        return idx

    return lax.map(f, qc).reshape(M, k)


def _ln_relu(x, g, b):
    m = jnp.mean(x, axis=-1, keepdims=True)
    v = jnp.mean((x - m) ** 2, axis=-1, keepdims=True)
    return jnp.maximum((x - m) / jnp.sqrt(v + _EPS) * g + b, 0.0)


def _mlp_body(qe_ref, gx_ref, gf_ref,
              w0a_ref, w0b_ref, b0_ref, g0_ref, be0_ref,
              w1_ref, b1_ref, g1_ref, be1_ref,
              a1a_ref, a1b_ref, ab1_ref, a2_ref,
              pw_ref, pb_ref, ab2s_ref,
              out_ref):
    rel = gx_ref[...] - qe_ref[...]
    ff = gf_ref[...]

    w0a = w0a_ref[...]
    xw = rel[:, 0:1] * w0a[0:1, :]
    xw = xw + rel[:, 1:2] * w0a[1:2, :]
    xw = xw + rel[:, 2:3] * w0a[2:3, :]
    xw = xw + jnp.dot(ff, w0b_ref[...], preferred_element_type=jnp.float32)
    xw = xw + b0_ref[...]
    x = _ln_relu(xw, g0_ref[...], be0_ref[...])
    x = jnp.dot(x, w1_ref[...], preferred_element_type=jnp.float32) + b1_ref[...]
    x = _ln_relu(x, g1_ref[...], be1_ref[...])

    a1a = a1a_ref[...]
    a = rel[:, 0:1] * a1a[0:1, :]
    a = a + rel[:, 1:2] * a1a[1:2, :]
    a = a + rel[:, 2:3] * a1a[2:3, :]
    a = a + jnp.dot(x, a1b_ref[...], preferred_element_type=jnp.float32)
    a = jnp.maximum(a + ab1_ref[...], 0.0)
    s = jnp.dot(a, a2_ref[...], preferred_element_type=jnp.float32)
    s = s + ab2s_ref[0]

    s3 = s.reshape(_QB, _NS, 1)
    m3 = jnp.max(s3, axis=1, keepdims=True)
    e3 = jnp.exp(s3 - m3)
    w3 = e3 / jnp.sum(e3, axis=1, keepdims=True)
    x3 = x.reshape(_QB, _NS, 128)
    feats = jnp.sum(x3 * w3, axis=1)
    out_ref[...] = jnp.dot(feats, pw_ref[...],
                           preferred_element_type=jnp.float32) + pb_ref[...]


def _mlp_attn(qxyz, gxyz, gfeat, W0, b0, g0, be0, W1, b1, g1, be1,
              aW1, ab1, aW2, ab2, pW, pb):
    n = _NPOINT * _NS
    qe = jnp.repeat(qxyz, _NS, axis=0)
    qe4 = jnp.pad(qe, ((0, 0), (0, 1)))
    gx4 = jnp.pad(gxyz, ((0, 0), (0, 1)))
    w0a = jnp.pad(W0[:3], ((0, 1), (0, 0)))
    w0b = W0[3:]
    a1a = jnp.pad(aW1[:3], ((0, 1), (0, 0)))
    a1b = aW1[3:]
    b2 = jnp.broadcast_to

    grid = _NPOINT // _QB
    blk = _QB * _NS
    wspec = lambda shape: pl.BlockSpec(shape, lambda i: (0, 0))
    out = pl.pallas_call(
        _mlp_body,
        grid=(grid,),
        in_specs=[
            pl.BlockSpec((blk, 4), lambda i: (i, 0)),
            pl.BlockSpec((blk, 4), lambda i: (i, 0)),
            pl.BlockSpec((blk, 128), lambda i: (i, 0)),
            wspec((4, 128)), wspec((128, 128)),
            wspec((1, 128)), wspec((1, 128)), wspec((1, 128)),
            wspec((128, 128)), wspec((1, 128)), wspec((1, 128)), wspec((1, 128)),
            wspec((4, 128)), wspec((128, 128)), wspec((1, 128)),
            wspec((128, 1)),
            wspec((128, 128)), wspec((1, 128)),
            pl.BlockSpec(memory_space=pltpu.SMEM),
        ],
        out_specs=pl.BlockSpec((_QB, 128), lambda i: (i, 0)),
        out_shape=jax.ShapeDtypeStruct((_NPOINT, 128), jnp.float32),
    )(qe4, gx4, gfeat,
      w0a, w0b, b0[None, :], g0[None, :], be0[None, :],
      W1, b1[None, :], g1[None, :], be1[None, :],
      a1a, a1b, ab1[None, :],
      aW2,
      pW, pb[None, :], ab2)
    return out


def kernel(xyz, features, offset, W0, b0, g0, be0, W1, b1, g1, be1,
           aW1, ab1, aW2, ab2, pW, pb):
    pads = _P - _N
    planes = [jnp.pad(xyz[:, k], (0, pads), constant_values=1e6).reshape(_R, 128)
              for k in range(3)]
    nx, ny, nz = _fps_pallas(planes)
    new_xyz = jnp.stack([nx, ny, nz], axis=-1)
    d = _knn_phase1(nx, ny, nz,
                    planes[0].reshape(1, _P), planes[1].reshape(1, _P),
                    planes[2].reshape(1, _P))
    _, nidx = lax.top_k(-d, _NS)
    gxyz = xyz[nidx].reshape(_NPOINT * _NS, 3)
    gfeat = features[nidx].reshape(_NPOINT * _NS, 128)
    out = _mlp_attn(new_xyz, gxyz, gfeat, W0, b0, g0, be0,
                    W1, b1, g1, be1, aW1, ab1, aW2, ab2, pW, pb)
    new_offset = jnp.array([_NPOINT], dtype=jnp.int32)
    return new_xyz, out, new_offset
